# Initial kernel scaffold; baseline (speedup 1.0000x reference)
import jax
import jax.numpy as jnp
from jax.experimental import pallas as pl


def kernel(x, edge_index, w0_0, b0_0, w1_0, b1_0, w0_1, b0_1, w1_1, b1_1, w0_2, b0_2, w1_2, b1_2, w0_3, b0_3, w1_3, b1_3, w0_4, b0_4, w1_4, b1_4):
    raise NotImplementedError("write your pallas kernel here")


# speedup vs baseline: 1.2398x; 1.2398x over previous
import functools

import jax
import jax.numpy as jnp
from jax import lax
from jax.experimental import pallas as pl
from jax.experimental.pallas import tpu as pltpu
from jax.experimental.pallas import tpu_sc as plsc

N = 1000
NPAD = 1024
E = 16000
EPAD = 16384
D = 256
LANES = 16
NTILES = 32
EPT = EPAD // NTILES
GCH = 128
SEG = 64
RPT = NPAD // NTILES
BE = 2048

_mesh = plsc.VectorSubcoreMesh(core_axis_name="c", subcore_axis_name="s")


def _edge_mlp_body(xd_ref, xs_ref, wa_ref, wb_ref, w1t_ref, b0_ref, b1_ref, h_ref):
    xd = xd_ref[...]
    diff = xs_ref[...] - xd
    h1 = jnp.dot(xd, wa_ref[...], preferred_element_type=jnp.float32)
    h1 = h1 + jnp.dot(diff, wb_ref[...], preferred_element_type=jnp.float32)
    h1 = jnp.maximum(h1 + b0_ref[...], 0.0)
    h_ref[...] = jnp.dot(h1, w1t_ref[...],
                         preferred_element_type=jnp.float32) + b1_ref[...]


def _edge_mlp(xd, xs, wa, wb, w1t, b0, b1):
    grid = (EPAD // BE,)
    return pl.pallas_call(
        _edge_mlp_body,
        grid=grid,
        in_specs=[
            pl.BlockSpec((BE, D), lambda i: (i, 0)),
            pl.BlockSpec((BE, D), lambda i: (i, 0)),
            pl.BlockSpec((D, D), lambda i: (0, 0)),
            pl.BlockSpec((D, D), lambda i: (0, 0)),
            pl.BlockSpec((D, D), lambda i: (0, 0)),
            pl.BlockSpec((1, D), lambda i: (0, 0)),
            pl.BlockSpec((1, D), lambda i: (0, 0)),
        ],
        out_specs=pl.BlockSpec((BE, D), lambda i: (i, 0)),
        out_shape=jax.ShapeDtypeStruct((EPAD, D), jnp.float32),
    )(xd, xs, wa, wb, w1t, b0, b1)


@functools.partial(
    pl.kernel,
    out_type=(
        jax.ShapeDtypeStruct((EPAD, D), jnp.float32),
        jax.ShapeDtypeStruct((EPAD, D), jnp.float32),
    ),
    mesh=_mesh,
    scratch_types=[
        pltpu.VMEM((EPT // GCH, GCH), jnp.int32),
        pltpu.VMEM((EPT // GCH, GCH), jnp.int32),
        pltpu.VMEM((GCH, D), jnp.float32),
        pltpu.VMEM((GCH, D), jnp.float32),
        pltpu.SemaphoreType.DMA,
        pltpu.SemaphoreType.DMA,
    ],
)
def _sc_gather(u_hbm, v_hbm, didx_hbm, sidx_hbm, gd_hbm, gs_hbm,
               didx_v, sidx_v, bufu, bufv, semu, semv):
    wid = lax.axis_index("s") * 2 + lax.axis_index("c")
    row0 = wid * (EPT // GCH)
    pltpu.sync_copy(didx_hbm.at[pl.ds(row0, EPT // GCH)], didx_v)
    pltpu.sync_copy(sidx_hbm.at[pl.ds(row0, EPT // GCH)], sidx_v)
    for j in range(EPT // GCH):
        cu = pltpu.async_copy(u_hbm.at[didx_v.at[j]], bufu, semu)
        cv = pltpu.async_copy(v_hbm.at[sidx_v.at[j]], bufv, semv)
        cu.wait()
        cv.wait()
        base = wid * EPT + j * GCH
        pltpu.sync_copy(bufu, gd_hbm.at[pl.ds(base, GCH)])
        pltpu.sync_copy(bufv, gs_hbm.at[pl.ds(base, GCH)])


@functools.partial(
    pl.kernel,
    out_type=jax.ShapeDtypeStruct((NPAD, D), jnp.float32),
    mesh=_mesh,
    scratch_types=[
        pltpu.VMEM((48,), jnp.int32),
        pltpu.VMEM((SEG + LANES,), jnp.int32),
        pltpu.VMEM((SEG, D), jnp.float32),
        pltpu.VMEM((RPT + 16, D), jnp.float32),
        pltpu.SemaphoreType.DMA,
        pltpu.SemaphoreType.DMA,
    ],
)
def _sc_segmax(h_hbm, sdst_hbm, starts_hbm, y_hbm,
               starts_v, dbuf, hbuf, agg, semd, semh):
    wid = lax.axis_index("s") * 2 + lax.axis_index("c")
    pltpu.sync_copy(starts_hbm, starts_v)

    zeros = jnp.zeros((LANES,), jnp.float32)

    @pl.loop(0, RPT + 16)
    def _(r):
        for k in range(D // LANES):
            agg[r, pl.ds(k * LANES, LANES)] = zeros

    a = starts_v[pl.ds(wid, LANES)][0]
    b = starts_v[pl.ds(wid + 1, LANES)][0]
    a0 = (a // SEG) * SEG
    nch = (b - a0 + SEG - 1) // SEG
    base_node = wid * RPT

    @pl.loop(0, nch)
    def _(j):
        off = a0 + j * SEG
        cd = pltpu.async_copy(sdst_hbm.at[pl.ds(off, SEG)], dbuf.at[pl.ds(0, SEG)], semd)
        ch = pltpu.async_copy(h_hbm.at[pl.ds(off, SEG)], hbuf, semh)
        cd.wait()
        ch.wait()

        @pl.loop(0, SEG)
        def _(i):
            r = dbuf[pl.ds(i, LANES)][0] - base_node
            rc = jnp.minimum(jnp.maximum(r, -1), RPT) + 8
            for k in range(D // LANES):
                sl = pl.ds(k * LANES, LANES)
                agg[rc, sl] = jnp.maximum(agg[rc, sl], hbuf[i, sl])

    pltpu.sync_copy(agg.at[pl.ds(8, RPT)], y_hbm.at[pl.ds(wid * RPT, RPT)])


def kernel(x, edge_index, w0_0, b0_0, w1_0, b1_0, w0_1, b0_1, w1_1, b1_1,
           w0_2, b0_2, w1_2, b1_2, w0_3, b0_3, w1_3, b1_3, w0_4, b0_4, w1_4, b1_4):
    src = edge_index[0].astype(jnp.int32)
    dst = edge_index[1].astype(jnp.int32)
    sdst, ssrc = lax.sort((dst, src), num_keys=1)
    starts = jnp.searchsorted(
        sdst, jnp.arange(0, NPAD + RPT, RPT, dtype=jnp.int32)).astype(jnp.int32)
    starts48 = jnp.zeros((48,), jnp.int32).at[:starts.shape[0]].set(starts)
    pad0 = jnp.zeros((EPAD - E,), jnp.int32)
    didx = jnp.concatenate([sdst, pad0]).reshape(EPAD // GCH, GCH)
    sidx = jnp.concatenate([ssrc, pad0]).reshape(EPAD // GCH, GCH)
    sdstseg = jnp.concatenate([sdst, jnp.full((EPAD - E,), 4 * NPAD, jnp.int32)])

    y = jnp.zeros((NPAD, D), jnp.float32).at[:N].set(x)
    params = [(w0_0, b0_0, w1_0, b1_0), (w0_1, b0_1, w1_1, b1_1),
              (w0_2, b0_2, w1_2, b1_2), (w0_3, b0_3, w1_3, b1_3),
              (w0_4, b0_4, w1_4, b1_4)]
    for (w0, b0, w1, b1) in params:
        wa = w0[:, :D].T
        wb = w0[:, D:].T
        xd, xs = _sc_gather(y, y, didx, sidx)
        h = _edge_mlp(xd, xs, wa, wb, w1.T, b0.reshape(1, D), b1.reshape(1, D))
        y = _sc_segmax(h, sdstseg, starts48)
    return y[:N]


# speedup vs baseline: 1.4269x; 1.1509x over previous
import functools

import jax
import jax.numpy as jnp
from jax import lax
from jax.experimental import pallas as pl
from jax.experimental.pallas import tpu as pltpu
from jax.experimental.pallas import tpu_sc as plsc

N = 1000
NPAD = 1024
E = 16000
EPAD = 16384
D = 256
LANES = 16
NTILES = 32
EPT = EPAD // NTILES
CH = 64
NCH = EPT // CH
SEG = 64
RPT = NPAD // NTILES
BE = 2048

_mesh = plsc.VectorSubcoreMesh(core_axis_name="c", subcore_axis_name="s")


def _edge_mlp_body(xd_ref, xs_ref, wa_ref, wb_ref, w1t_ref, b0_ref, b1_ref, h_ref):
    xd = xd_ref[...]
    diff = xs_ref[...] - xd
    h1 = jnp.dot(xd, wa_ref[...], preferred_element_type=jnp.float32)
    h1 = h1 + jnp.dot(diff, wb_ref[...], preferred_element_type=jnp.float32)
    h1 = jnp.maximum(h1 + b0_ref[...], 0.0)
    h_ref[...] = jnp.dot(h1, w1t_ref[...],
                         preferred_element_type=jnp.float32) + b1_ref[...]


def _edge_mlp(xd, xs, wa, wb, w1t, b0, b1):
    grid = (EPAD // BE,)
    return pl.pallas_call(
        _edge_mlp_body,
        grid=grid,
        in_specs=[
            pl.BlockSpec((BE, D), lambda i: (i, 0)),
            pl.BlockSpec((BE, D), lambda i: (i, 0)),
            pl.BlockSpec((D, D), lambda i: (0, 0)),
            pl.BlockSpec((D, D), lambda i: (0, 0)),
            pl.BlockSpec((D, D), lambda i: (0, 0)),
            pl.BlockSpec((1, D), lambda i: (0, 0)),
            pl.BlockSpec((1, D), lambda i: (0, 0)),
        ],
        out_specs=pl.BlockSpec((BE, D), lambda i: (i, 0)),
        out_shape=jax.ShapeDtypeStruct((EPAD, D), jnp.float32),
    )(xd, xs, wa, wb, w1t, b0, b1)


@functools.partial(
    pl.kernel,
    out_type=(
        jax.ShapeDtypeStruct((EPAD, D), jnp.float32),
        jax.ShapeDtypeStruct((EPAD, D), jnp.float32),
    ),
    mesh=_mesh,
    scratch_types=[
        pltpu.VMEM((NCH, CH), jnp.int32),
        pltpu.VMEM((NCH, CH), jnp.int32),
        pltpu.VMEM((CH, D), jnp.float32),
        pltpu.VMEM((CH, D), jnp.float32),
        pltpu.VMEM((CH, D), jnp.float32),
        pltpu.VMEM((CH, D), jnp.float32),
        pltpu.SemaphoreType.DMA,
        pltpu.SemaphoreType.DMA,
        pltpu.SemaphoreType.DMA,
        pltpu.SemaphoreType.DMA,
        pltpu.SemaphoreType.DMA,
        pltpu.SemaphoreType.DMA,
        pltpu.SemaphoreType.DMA,
        pltpu.SemaphoreType.DMA,
    ],
)
def _sc_gather(y1_hbm, y2_hbm, didx_hbm, sidx_hbm, gd_hbm, gs_hbm,
               didx_v, sidx_v, bufu0, bufu1, bufv0, bufv1,
               gu0, gu1, gv0, gv1, wu0, wu1, wv0, wv1):
    wid = lax.axis_index("s") * 2 + lax.axis_index("c")
    row0 = wid * NCH
    pltpu.sync_copy(didx_hbm.at[pl.ds(row0, NCH)], didx_v)
    pltpu.sync_copy(sidx_hbm.at[pl.ds(row0, NCH)], sidx_v)
    bufu = (bufu0, bufu1)
    bufv = (bufv0, bufv1)
    gu = (gu0, gu1)
    gv = (gv0, gv1)
    wu = (wu0, wu1)
    wv = (wv0, wv1)
    gwaits = {}
    wwaits = {}
    gwaits[0] = (pltpu.async_copy(y1_hbm.at[didx_v.at[0]], bufu[0], gu[0]),
                 pltpu.async_copy(y2_hbm.at[sidx_v.at[0]], bufv[0], gv[0]))
    for j in range(NCH):
        b = j & 1
        nb = b ^ 1
        if j + 1 < NCH:
            if j >= 1:
                cu, cv = wwaits.pop(j - 1)
                cu.wait()
                cv.wait()
            gwaits[j + 1] = (
                pltpu.async_copy(y1_hbm.at[didx_v.at[j + 1]], bufu[nb], gu[nb]),
                pltpu.async_copy(y2_hbm.at[sidx_v.at[j + 1]], bufv[nb], gv[nb]))
        cu, cv = gwaits.pop(j)
        cu.wait()
        cv.wait()
        base = wid * EPT + j * CH
        wwaits[j] = (
            pltpu.async_copy(bufu[b], gd_hbm.at[pl.ds(base, CH)], wu[b]),
            pltpu.async_copy(bufv[b], gs_hbm.at[pl.ds(base, CH)], wv[b]))
    for j in (NCH - 2, NCH - 1):
        cu, cv = wwaits.pop(j)
        cu.wait()
        cv.wait()


@functools.partial(
    pl.kernel,
    out_type=jax.ShapeDtypeStruct((NPAD, D), jnp.float32),
    mesh=_mesh,
    scratch_types=[
        pltpu.VMEM((48,), jnp.int32),
        pltpu.VMEM((SEG,), jnp.int32),
        pltpu.VMEM((SEG,), jnp.int32),
        pltpu.VMEM((SEG, D), jnp.float32),
        pltpu.VMEM((SEG, D), jnp.float32),
        pltpu.VMEM((RPT + 16, D), jnp.float32),
        pltpu.SemaphoreType.DMA,
        pltpu.SemaphoreType.DMA,
        pltpu.SemaphoreType.DMA,
        pltpu.SemaphoreType.DMA,
    ],
)
def _sc_segmax(h_hbm, sdst_hbm, starts_hbm, y_hbm,
               starts_v, dbuf0, dbuf1, hbuf0, hbuf1, agg, sd0, sd1, sh0, sh1):
    wid = lax.axis_index("s") * 2 + lax.axis_index("c")
    pltpu.sync_copy(starts_hbm, starts_v)

    zeros = jnp.zeros((LANES,), jnp.float32)

    @pl.loop(0, RPT + 16)
    def _(r):
        for k in range(D // LANES):
            agg[r, pl.ds(k * LANES, LANES)] = zeros

    a = starts_v[pl.ds(wid, LANES)][0]
    b = starts_v[pl.ds(wid + 1, LANES)][0]
    a0 = (a // SEG) * SEG
    nch = (b - a0 + SEG - 1) // SEG
    end64 = a0 + SEG * nch
    base_node = wid * RPT

    def _issue(off, dbuf, hbuf, sdm, shm):
        pltpu.async_copy(sdst_hbm.at[pl.ds(off, SEG)], dbuf, sdm)
        pltpu.async_copy(h_hbm.at[pl.ds(off, SEG)], hbuf, shm)

    def _wait(dbuf, hbuf, sdm, shm):
        pltpu.make_async_copy(sdst_hbm.at[pl.ds(0, SEG)], dbuf, sdm).wait()
        pltpu.make_async_copy(h_hbm.at[pl.ds(0, SEG)], hbuf, shm).wait()

    def _accum(dbuf, hbuf):
        @pl.loop(0, SEG // LANES)
        def _(g):
            dvec = dbuf[pl.ds(g * LANES, LANES)]
            rcvec = jnp.minimum(jnp.maximum(dvec - base_node, -1), RPT) + 8
            for i in range(LANES):
                rc = rcvec[i]
                for k in range(D // LANES):
                    sl = pl.ds(k * LANES, LANES)
                    agg[rc, sl] = jnp.maximum(
                        agg[rc, sl], hbuf[g * LANES + i, sl])

    @pl.when(nch > 0)
    def _():
        _issue(a0, dbuf0, hbuf0, sd0, sh0)

    @pl.loop(0, (nch + 1) // 2)
    def _(t):
        off0 = a0 + t * (2 * SEG)

        @pl.when(off0 + SEG < end64)
        def _():
            _issue(off0 + SEG, dbuf1, hbuf1, sd1, sh1)

        _wait(dbuf0, hbuf0, sd0, sh0)
        _accum(dbuf0, hbuf0)

        @pl.when(off0 + 2 * SEG < end64)
        def _():
            _issue(off0 + 2 * SEG, dbuf0, hbuf0, sd0, sh0)

        @pl.when(off0 + SEG < end64)
        def _():
            _wait(dbuf1, hbuf1, sd1, sh1)
            _accum(dbuf1, hbuf1)

    pltpu.sync_copy(agg.at[pl.ds(8, RPT)], y_hbm.at[pl.ds(wid * RPT, RPT)])


def kernel(x, edge_index, w0_0, b0_0, w1_0, b1_0, w0_1, b0_1, w1_1, b1_1,
           w0_2, b0_2, w1_2, b1_2, w0_3, b0_3, w1_3, b1_3, w0_4, b0_4, w1_4, b1_4):
    src = edge_index[0].astype(jnp.int32)
    dst = edge_index[1].astype(jnp.int32)
    sdst, ssrc = lax.sort((dst, src), num_keys=1)
    starts = jnp.searchsorted(
        sdst, jnp.arange(0, NPAD + RPT, RPT, dtype=jnp.int32)).astype(jnp.int32)
    starts48 = jnp.zeros((48,), jnp.int32).at[:starts.shape[0]].set(starts)
    pad0 = jnp.zeros((EPAD - E,), jnp.int32)
    didx = jnp.concatenate([sdst, pad0]).reshape(EPAD // CH, CH)
    sidx = jnp.concatenate([ssrc, pad0]).reshape(EPAD // CH, CH)
    sdstseg = jnp.concatenate([sdst, jnp.full((EPAD - E,), 4 * NPAD, jnp.int32)])

    y = jnp.zeros((NPAD, D), jnp.float32).at[:N].set(x)
    params = [(w0_0, b0_0, w1_0, b1_0), (w0_1, b0_1, w1_1, b1_1),
              (w0_2, b0_2, w1_2, b1_2), (w0_3, b0_3, w1_3, b1_3),
              (w0_4, b0_4, w1_4, b1_4)]
    for (w0, b0, w1, b1) in params:
        wa = w0[:, :D].T
        wb = w0[:, D:].T
        xd, xs = _sc_gather(y, y, didx, sidx)
        h = _edge_mlp(xd, xs, wa, wb, w1.T, b0.reshape(1, D), b1.reshape(1, D))
        y = _sc_segmax(h, sdstseg, starts48)
    return y[:N]


# speedup vs baseline: 1.6055x; 1.1252x over previous
import functools

import jax
import jax.numpy as jnp
from jax import lax
from jax.experimental import pallas as pl
from jax.experimental.pallas import tpu as pltpu
from jax.experimental.pallas import tpu_sc as plsc

N = 1000
NPAD = 1024
E = 16000
EPAD = 16384
D = 256
LANES = 16
NTILES = 32
EPT = EPAD // NTILES
CH = 64
NCH = EPT // CH
SEG = 128
RPT = NPAD // NTILES
BE = 2048

_mesh = plsc.VectorSubcoreMesh(core_axis_name="c", subcore_axis_name="s")


def _edge_mlp_body(xd_ref, xs_ref, wa_ref, wb_ref, w1t_ref, b0_ref, b1_ref, h_ref):
    xd = xd_ref[...]
    diff = xs_ref[...] - xd
    h1 = jnp.dot(xd, wa_ref[...], preferred_element_type=jnp.float32)
    h1 = h1 + jnp.dot(diff, wb_ref[...], preferred_element_type=jnp.float32)
    h1 = jnp.maximum(h1 + b0_ref[...], 0.0)
    h_ref[...] = jnp.dot(h1, w1t_ref[...],
                         preferred_element_type=jnp.float32) + b1_ref[...]


def _edge_mlp(xd, xs, wa, wb, w1t, b0, b1):
    grid = (EPAD // BE,)
    return pl.pallas_call(
        _edge_mlp_body,
        grid=grid,
        in_specs=[
            pl.BlockSpec((BE, D), lambda i: (i, 0)),
            pl.BlockSpec((BE, D), lambda i: (i, 0)),
            pl.BlockSpec((D, D), lambda i: (0, 0)),
            pl.BlockSpec((D, D), lambda i: (0, 0)),
            pl.BlockSpec((D, D), lambda i: (0, 0)),
            pl.BlockSpec((1, D), lambda i: (0, 0)),
            pl.BlockSpec((1, D), lambda i: (0, 0)),
        ],
        out_specs=pl.BlockSpec((BE, D), lambda i: (i, 0)),
        out_shape=jax.ShapeDtypeStruct((EPAD, D), jnp.float32),
    )(xd, xs, wa, wb, w1t, b0, b1)


@functools.partial(
    pl.kernel,
    out_type=(
        jax.ShapeDtypeStruct((EPAD, D), jnp.float32),
        jax.ShapeDtypeStruct((EPAD, D), jnp.float32),
    ),
    mesh=_mesh,
    scratch_types=[
        pltpu.VMEM((NCH, CH), jnp.int32),
        pltpu.VMEM((NCH, CH), jnp.int32),
        pltpu.VMEM((CH, D), jnp.float32),
        pltpu.VMEM((CH, D), jnp.float32),
        pltpu.VMEM((CH, D), jnp.float32),
        pltpu.VMEM((CH, D), jnp.float32),
        pltpu.SemaphoreType.DMA,
        pltpu.SemaphoreType.DMA,
        pltpu.SemaphoreType.DMA,
        pltpu.SemaphoreType.DMA,
        pltpu.SemaphoreType.DMA,
        pltpu.SemaphoreType.DMA,
        pltpu.SemaphoreType.DMA,
        pltpu.SemaphoreType.DMA,
    ],
)
def _sc_gather(y1_hbm, y2_hbm, didx_hbm, sidx_hbm, gd_hbm, gs_hbm,
               didx_v, sidx_v, bufu0, bufu1, bufv0, bufv1,
               gu0, gu1, gv0, gv1, wu0, wu1, wv0, wv1):
    wid = lax.axis_index("s") * 2 + lax.axis_index("c")
    row0 = wid * NCH
    pltpu.sync_copy(didx_hbm.at[pl.ds(row0, NCH)], didx_v)
    pltpu.sync_copy(sidx_hbm.at[pl.ds(row0, NCH)], sidx_v)
    bufu = (bufu0, bufu1)
    bufv = (bufv0, bufv1)
    gu = (gu0, gu1)
    gv = (gv0, gv1)
    wu = (wu0, wu1)
    wv = (wv0, wv1)
    gwaits = {}
    wwaits = {}
    gwaits[0] = (pltpu.async_copy(y1_hbm.at[didx_v.at[0]], bufu[0], gu[0]),
                 pltpu.async_copy(y2_hbm.at[sidx_v.at[0]], bufv[0], gv[0]))
    for j in range(NCH):
        b = j & 1
        nb = b ^ 1
        if j + 1 < NCH:
            if j >= 1:
                cu, cv = wwaits.pop(j - 1)
                cu.wait()
                cv.wait()
            gwaits[j + 1] = (
                pltpu.async_copy(y1_hbm.at[didx_v.at[j + 1]], bufu[nb], gu[nb]),
                pltpu.async_copy(y2_hbm.at[sidx_v.at[j + 1]], bufv[nb], gv[nb]))
        cu, cv = gwaits.pop(j)
        cu.wait()
        cv.wait()
        base = wid * EPT + j * CH
        wwaits[j] = (
            pltpu.async_copy(bufu[b], gd_hbm.at[pl.ds(base, CH)], wu[b]),
            pltpu.async_copy(bufv[b], gs_hbm.at[pl.ds(base, CH)], wv[b]))
    for j in (NCH - 2, NCH - 1):
        cu, cv = wwaits.pop(j)
        cu.wait()
        cv.wait()


@functools.partial(
    pl.kernel,
    out_type=jax.ShapeDtypeStruct((NPAD, D), jnp.float32),
    mesh=_mesh,
    scratch_types=[
        pltpu.VMEM((48,), jnp.int32),
        pltpu.VMEM((SEG + LANES,), jnp.int32),
        pltpu.VMEM((SEG + LANES,), jnp.int32),
        pltpu.VMEM((SEG, D), jnp.float32),
        pltpu.VMEM((SEG, D), jnp.float32),
        pltpu.VMEM((RPT + 16, D), jnp.float32),
        pltpu.SemaphoreType.DMA,
        pltpu.SemaphoreType.DMA,
        pltpu.SemaphoreType.DMA,
        pltpu.SemaphoreType.DMA,
    ],
)
def _sc_segmax(h_hbm, sdst_hbm, starts_hbm, y_hbm,
               starts_v, dbuf0, dbuf1, hbuf0, hbuf1, agg, sd0, sd1, sh0, sh1):
    wid = lax.axis_index("s") * 2 + lax.axis_index("c")
    pltpu.sync_copy(starts_hbm, starts_v)

    zeros = jnp.zeros((LANES,), jnp.float32)

    @pl.loop(0, RPT + 16)
    def _(r):
        for k in range(D // LANES):
            agg[r, pl.ds(k * LANES, LANES)] = zeros

    a = starts_v[pl.ds(wid, LANES)][0]
    b = starts_v[pl.ds(wid + 1, LANES)][0]
    a0 = (a // SEG) * SEG
    nch = (b - a0 + SEG - 1) // SEG
    end64 = a0 + SEG * nch
    base_node = wid * RPT

    def _issue(off, dbuf, hbuf, sdm, shm):
        pltpu.async_copy(sdst_hbm.at[pl.ds(off, SEG)], dbuf.at[pl.ds(0, SEG)], sdm)
        pltpu.async_copy(h_hbm.at[pl.ds(off, SEG)], hbuf, shm)

    def _wait(dbuf, hbuf, sdm, shm):
        pltpu.make_async_copy(sdst_hbm.at[pl.ds(0, SEG)],
                              dbuf.at[pl.ds(0, SEG)], sdm).wait()
        pltpu.make_async_copy(h_hbm.at[pl.ds(0, SEG)], hbuf, shm).wait()

    NV = D // LANES
    neg = jnp.full((LANES,), -3.0e38, jnp.float32)

    def _flush(row, accs):
        for k in range(NV):
            sl = pl.ds(k * LANES, LANES)
            agg[row, sl] = jnp.maximum(agg[row, sl], accs[k])

    def _accum(dbuf, hbuf):
        def body(i, carry):
            cur = carry[0]
            accs = carry[1:]
            dv = dbuf[pl.ds(i, LANES)][0]
            rc = jnp.minimum(jnp.maximum(dv - base_node, -1), RPT) + 8
            nb = rc != cur

            @pl.when(nb)
            def _():
                _flush(cur, accs)

            hrows = [hbuf[i, pl.ds(k * LANES, LANES)] for k in range(NV)]
            new = tuple(
                jnp.where(nb, hrows[k], jnp.maximum(accs[k], hrows[k]))
                for k in range(NV))
            return (rc,) + new

        final = lax.fori_loop(0, SEG, body, (jnp.int32(7),) + (neg,) * NV)
        _flush(final[0], final[1:])

    @pl.when(nch > 0)
    def _():
        _issue(a0, dbuf0, hbuf0, sd0, sh0)

    @pl.loop(0, (nch + 1) // 2)
    def _(t):
        off0 = a0 + t * (2 * SEG)

        @pl.when(off0 + SEG < end64)
        def _():
            _issue(off0 + SEG, dbuf1, hbuf1, sd1, sh1)

        _wait(dbuf0, hbuf0, sd0, sh0)
        _accum(dbuf0, hbuf0)

        @pl.when(off0 + 2 * SEG < end64)
        def _():
            _issue(off0 + 2 * SEG, dbuf0, hbuf0, sd0, sh0)

        @pl.when(off0 + SEG < end64)
        def _():
            _wait(dbuf1, hbuf1, sd1, sh1)
            _accum(dbuf1, hbuf1)

    pltpu.sync_copy(agg.at[pl.ds(8, RPT)], y_hbm.at[pl.ds(wid * RPT, RPT)])


def kernel(x, edge_index, w0_0, b0_0, w1_0, b1_0, w0_1, b0_1, w1_1, b1_1,
           w0_2, b0_2, w1_2, b1_2, w0_3, b0_3, w1_3, b1_3, w0_4, b0_4, w1_4, b1_4):
    src = edge_index[0].astype(jnp.int32)
    dst = edge_index[1].astype(jnp.int32)
    sdst, ssrc = lax.sort((dst, src), num_keys=1)
    starts = jnp.searchsorted(
        sdst, jnp.arange(0, NPAD + RPT, RPT, dtype=jnp.int32)).astype(jnp.int32)
    starts48 = jnp.zeros((48,), jnp.int32).at[:starts.shape[0]].set(starts)
    pad0 = jnp.zeros((EPAD - E,), jnp.int32)
    didx = jnp.concatenate([sdst, pad0]).reshape(EPAD // CH, CH)
    sidx = jnp.concatenate([ssrc, pad0]).reshape(EPAD // CH, CH)
    sdstseg = jnp.concatenate([sdst, jnp.full((EPAD - E,), 4 * NPAD, jnp.int32)])

    y = jnp.zeros((NPAD, D), jnp.float32).at[:N].set(x)
    params = [(w0_0, b0_0, w1_0, b1_0), (w0_1, b0_1, w1_1, b1_1),
              (w0_2, b0_2, w1_2, b1_2), (w0_3, b0_3, w1_3, b1_3),
              (w0_4, b0_4, w1_4, b1_4)]
    for (w0, b0, w1, b1) in params:
        wa = w0[:, :D].T
        wb = w0[:, D:].T
        xd, xs = _sc_gather(y, y, didx, sidx)
        h = _edge_mlp(xd, xs, wa, wb, w1.T, b0.reshape(1, D), b1.reshape(1, D))
        y = _sc_segmax(h, sdstseg, starts48)
    return y[:N]


# speedup vs baseline: 1.9149x; 1.1927x over previous
import functools

import jax
import jax.numpy as jnp
from jax import lax
from jax.experimental import pallas as pl
from jax.experimental.pallas import tpu as pltpu
from jax.experimental.pallas import tpu_sc as plsc

N = 1000
NPAD = 1024
E = 16000
EPAD = 16384
D = 256
LANES = 16
NTILES = 32
EPT = EPAD // NTILES
CH = 64
NCH = EPT // CH
SEG = 128
RPT = NPAD // NTILES
BE = 2048

_mesh = plsc.VectorSubcoreMesh(core_axis_name="c", subcore_axis_name="s")


def _edge_mlp_body(xd_ref, xs_ref, wa_ref, wb_ref, w1t_ref, b0_ref, b1_ref, h_ref):
    xd = xd_ref[...]
    diff = xs_ref[...] - xd
    h1 = jnp.dot(xd, wa_ref[...], preferred_element_type=jnp.float32)
    h1 = h1 + jnp.dot(diff, wb_ref[...], preferred_element_type=jnp.float32)
    h1 = jnp.maximum(h1 + b0_ref[...], 0.0)
    h_ref[...] = jnp.dot(h1, w1t_ref[...],
                         preferred_element_type=jnp.float32) + b1_ref[...]


def _edge_mlp(xd, xs, wa, wb, w1t, b0, b1):
    grid = (EPAD // BE,)
    return pl.pallas_call(
        _edge_mlp_body,
        grid=grid,
        in_specs=[
            pl.BlockSpec((BE, D), lambda i: (i, 0)),
            pl.BlockSpec((BE, D), lambda i: (i, 0)),
            pl.BlockSpec((D, D), lambda i: (0, 0)),
            pl.BlockSpec((D, D), lambda i: (0, 0)),
            pl.BlockSpec((D, D), lambda i: (0, 0)),
            pl.BlockSpec((1, D), lambda i: (0, 0)),
            pl.BlockSpec((1, D), lambda i: (0, 0)),
        ],
        out_specs=pl.BlockSpec((BE, D), lambda i: (i, 0)),
        out_shape=jax.ShapeDtypeStruct((EPAD, D), jnp.float32),
    )(xd, xs, wa, wb, w1t, b0, b1)


YW = 128
NV = D // LANES


@functools.partial(
    pl.kernel,
    out_type=(
        jax.ShapeDtypeStruct((EPAD, D), jnp.float32),
        jax.ShapeDtypeStruct((EPAD, D), jnp.float32),
    ),
    mesh=_mesh,
    scratch_types=[
        pltpu.VMEM((NCH, CH), jnp.int32),
        pltpu.VMEM((NCH, CH), jnp.int32),
        pltpu.VMEM((EPT + LANES,), jnp.int32),
        pltpu.VMEM((YW, D), jnp.float32),
        pltpu.VMEM((CH, D), jnp.float32),
        pltpu.VMEM((CH, D), jnp.float32),
        pltpu.VMEM((CH, D), jnp.float32),
        pltpu.VMEM((CH, D), jnp.float32),
        pltpu.SemaphoreType.DMA,
        pltpu.SemaphoreType.DMA,
        pltpu.SemaphoreType.DMA,
        pltpu.SemaphoreType.DMA,
        pltpu.SemaphoreType.DMA,
        pltpu.SemaphoreType.DMA,
        pltpu.SemaphoreType.DMA,
        pltpu.SemaphoreType.DMA,
    ],
)
def _sc_gather(y1_hbm, y2_hbm, didx_hbm, sidx_hbm, sdst1d_hbm, gd_hbm, gs_hbm,
               didx_v, sidx_v, dbuf1d, ybuf, bufu0, bufu1, bufv0, bufv1,
               gu0, gu1, gv0, gv1, wu0, wu1, wv0, wv1):
    wid = lax.axis_index("s") * 2 + lax.axis_index("c")
    row0 = wid * NCH
    base_e = wid * EPT
    pltpu.sync_copy(didx_hbm.at[pl.ds(row0, NCH)], didx_v)
    pltpu.sync_copy(sidx_hbm.at[pl.ds(row0, NCH)], sidx_v)
    pltpu.sync_copy(sdst1d_hbm.at[pl.ds(base_e, EPT + LANES)], dbuf1d)
    bufu = (bufu0, bufu1)
    bufv = (bufv0, bufv1)
    gu = (gu0, gu1)
    gv = (gv0, gv1)
    wu = (wu0, wu1)
    wv = (wv0, wv1)

    d_lo = dbuf1d[pl.ds(0, LANES)][0]
    d_hi = dbuf1d[pl.ds(EPT - 1, LANES)][0]
    w_lo = jnp.minimum((d_lo // 8) * 8, NPAD - YW)
    fast = (d_hi - w_lo) < YW

    srcg = {}
    srcg[0] = pltpu.async_copy(y2_hbm.at[sidx_v.at[0]], bufv[0], gv[0])

    @pl.when(fast)
    def _():
        pltpu.sync_copy(y1_hbm.at[pl.ds(w_lo, YW)], ybuf)

    for j in range(NCH):
        b = j & 1
        nb = b ^ 1
        if j + 1 < NCH:
            if j >= 1:
                pltpu.make_async_copy(
                    bufv[nb], gs_hbm.at[pl.ds(0, CH)], wv[nb]).wait()
            srcg[j + 1] = pltpu.async_copy(
                y2_hbm.at[sidx_v.at[j + 1]], bufv[nb], gv[nb])
        if j >= 2:
            pltpu.make_async_copy(
                bufu[b], gd_hbm.at[pl.ds(0, CH)], wu[b]).wait()

        @pl.when(fast)
        def _(jj=j, bb=b):
            @pl.loop(0, CH)
            def _(i):
                dv = dbuf1d[pl.ds(jj * CH + i, LANES)][0]
                lr = dv - w_lo
                for k in range(NV):
                    sl = pl.ds(k * LANES, LANES)
                    bufu[bb][i, sl] = ybuf[lr, sl]

        @pl.when(jnp.logical_not(fast))
        def _(jj=j, bb=b):
            pltpu.async_copy(y1_hbm.at[didx_v.at[jj]], bufu[bb], gu[bb]).wait()

        pltpu.async_copy(bufu[b], gd_hbm.at[pl.ds(base_e + j * CH, CH)], wu[b])
        srcg.pop(j).wait()
        pltpu.async_copy(bufv[b], gs_hbm.at[pl.ds(base_e + j * CH, CH)], wv[b])

    for b in (0, 1):
        pltpu.make_async_copy(bufu[b], gd_hbm.at[pl.ds(0, CH)], wu[b]).wait()
        pltpu.make_async_copy(bufv[b], gs_hbm.at[pl.ds(0, CH)], wv[b]).wait()


@functools.partial(
    pl.kernel,
    out_type=jax.ShapeDtypeStruct((NPAD, D), jnp.float32),
    mesh=_mesh,
    scratch_types=[
        pltpu.VMEM((48,), jnp.int32),
        pltpu.VMEM((SEG + LANES,), jnp.int32),
        pltpu.VMEM((SEG + LANES,), jnp.int32),
        pltpu.VMEM((SEG, D), jnp.float32),
        pltpu.VMEM((SEG, D), jnp.float32),
        pltpu.VMEM((RPT + 16, D), jnp.float32),
        pltpu.SemaphoreType.DMA,
        pltpu.SemaphoreType.DMA,
        pltpu.SemaphoreType.DMA,
        pltpu.SemaphoreType.DMA,
    ],
)
def _sc_segmax(h_hbm, sdst_hbm, starts_hbm, y_hbm,
               starts_v, dbuf0, dbuf1, hbuf0, hbuf1, agg, sd0, sd1, sh0, sh1):
    wid = lax.axis_index("s") * 2 + lax.axis_index("c")
    pltpu.sync_copy(starts_hbm, starts_v)

    zeros = jnp.zeros((LANES,), jnp.float32)

    @pl.loop(0, RPT + 16)
    def _(r):
        for k in range(D // LANES):
            agg[r, pl.ds(k * LANES, LANES)] = zeros

    a = starts_v[pl.ds(wid, LANES)][0]
    b = starts_v[pl.ds(wid + 1, LANES)][0]
    a0 = (a // SEG) * SEG
    nch = (b - a0 + SEG - 1) // SEG
    end64 = a0 + SEG * nch
    base_node = wid * RPT

    def _issue(off, dbuf, hbuf, sdm, shm):
        pltpu.async_copy(sdst_hbm.at[pl.ds(off, SEG)], dbuf.at[pl.ds(0, SEG)], sdm)
        pltpu.async_copy(h_hbm.at[pl.ds(off, SEG)], hbuf, shm)

    def _wait(dbuf, hbuf, sdm, shm):
        pltpu.make_async_copy(sdst_hbm.at[pl.ds(0, SEG)],
                              dbuf.at[pl.ds(0, SEG)], sdm).wait()
        pltpu.make_async_copy(h_hbm.at[pl.ds(0, SEG)], hbuf, shm).wait()

    NV = D // LANES
    neg = jnp.full((LANES,), -3.0e38, jnp.float32)

    def _flush(row, accs):
        for k in range(NV):
            sl = pl.ds(k * LANES, LANES)
            agg[row, sl] = jnp.maximum(agg[row, sl], accs[k])

    def _accum(dbuf, hbuf):
        def body(i, carry):
            cur = carry[0]
            accs = carry[1:]
            dv = dbuf[pl.ds(i, LANES)][0]
            rc = jnp.minimum(jnp.maximum(dv - base_node, -1), RPT) + 8
            nb = rc != cur

            @pl.when(nb)
            def _():
                _flush(cur, accs)

            hrows = [hbuf[i, pl.ds(k * LANES, LANES)] for k in range(NV)]
            new = tuple(
                jnp.where(nb, hrows[k], jnp.maximum(accs[k], hrows[k]))
                for k in range(NV))
            return (rc,) + new

        final = lax.fori_loop(0, SEG, body, (jnp.int32(7),) + (neg,) * NV)
        _flush(final[0], final[1:])

    @pl.when(nch > 0)
    def _():
        _issue(a0, dbuf0, hbuf0, sd0, sh0)

    @pl.loop(0, (nch + 1) // 2)
    def _(t):
        off0 = a0 + t * (2 * SEG)

        @pl.when(off0 + SEG < end64)
        def _():
            _issue(off0 + SEG, dbuf1, hbuf1, sd1, sh1)

        _wait(dbuf0, hbuf0, sd0, sh0)
        _accum(dbuf0, hbuf0)

        @pl.when(off0 + 2 * SEG < end64)
        def _():
            _issue(off0 + 2 * SEG, dbuf0, hbuf0, sd0, sh0)

        @pl.when(off0 + SEG < end64)
        def _():
            _wait(dbuf1, hbuf1, sd1, sh1)
            _accum(dbuf1, hbuf1)

    pltpu.sync_copy(agg.at[pl.ds(8, RPT)], y_hbm.at[pl.ds(wid * RPT, RPT)])


def kernel(x, edge_index, w0_0, b0_0, w1_0, b1_0, w0_1, b0_1, w1_1, b1_1,
           w0_2, b0_2, w1_2, b1_2, w0_3, b0_3, w1_3, b1_3, w0_4, b0_4, w1_4, b1_4):
    src = edge_index[0].astype(jnp.int32)
    dst = edge_index[1].astype(jnp.int32)
    sdst, ssrc = lax.sort((dst, src), num_keys=1)
    starts = jnp.searchsorted(
        sdst, jnp.arange(0, NPAD + RPT, RPT, dtype=jnp.int32)).astype(jnp.int32)
    starts48 = jnp.zeros((48,), jnp.int32).at[:starts.shape[0]].set(starts)
    pad0 = jnp.zeros((EPAD - E,), jnp.int32)
    didx = jnp.concatenate([sdst, pad0]).reshape(EPAD // CH, CH)
    sidx = jnp.concatenate([ssrc, pad0]).reshape(EPAD // CH, CH)
    sdstseg = jnp.concatenate(
        [sdst, jnp.full((EPAD - E + LANES,), NPAD - 8, jnp.int32)])

    y = jnp.zeros((NPAD, D), jnp.float32).at[:N].set(x)
    params = [(w0_0, b0_0, w1_0, b1_0), (w0_1, b0_1, w1_1, b1_1),
              (w0_2, b0_2, w1_2, b1_2), (w0_3, b0_3, w1_3, b1_3),
              (w0_4, b0_4, w1_4, b1_4)]
    for (w0, b0, w1, b1) in params:
        wa = w0[:, :D].T
        wb = w0[:, D:].T
        xd, xs = _sc_gather(y, y, didx, sidx, sdstseg)
        h = _edge_mlp(xd, xs, wa, wb, w1.T, b0.reshape(1, D), b1.reshape(1, D))
        y = _sc_segmax(h, sdstseg, starts48)
    return y[:N]


# speedup vs baseline: 1.9415x; 1.0139x over previous
import functools

import jax
import jax.numpy as jnp
from jax import lax
from jax.experimental import pallas as pl
from jax.experimental.pallas import tpu as pltpu
from jax.experimental.pallas import tpu_sc as plsc

N = 1000
NPAD = 1024
E = 16000
EPAD = 16384
D = 256
LANES = 16
NTILES = 32
EPT = EPAD // NTILES
CH = 64
NCH = EPT // CH
SEG = 128
RPT = NPAD // NTILES
BE = 2048

_mesh = plsc.VectorSubcoreMesh(core_axis_name="c", subcore_axis_name="s")


def _edge_mlp_body(xd_ref, xs_ref, wa_ref, wb_ref, w1t_ref, b0_ref, b1_ref, h_ref):
    xd = xd_ref[...]
    diff = xs_ref[...] - xd
    h1 = jnp.dot(xd, wa_ref[...], preferred_element_type=jnp.float32)
    h1 = h1 + jnp.dot(diff, wb_ref[...], preferred_element_type=jnp.float32)
    h1 = jnp.maximum(h1 + b0_ref[...], 0.0)
    h_ref[...] = jnp.dot(h1, w1t_ref[...],
                         preferred_element_type=jnp.float32) + b1_ref[...]


def _edge_mlp(xd, xs, wa, wb, w1t, b0, b1):
    grid = (EPAD // BE,)
    return pl.pallas_call(
        _edge_mlp_body,
        grid=grid,
        in_specs=[
            pl.BlockSpec((BE, D), lambda i: (i, 0)),
            pl.BlockSpec((BE, D), lambda i: (i, 0)),
            pl.BlockSpec((D, D), lambda i: (0, 0)),
            pl.BlockSpec((D, D), lambda i: (0, 0)),
            pl.BlockSpec((D, D), lambda i: (0, 0)),
            pl.BlockSpec((1, D), lambda i: (0, 0)),
            pl.BlockSpec((1, D), lambda i: (0, 0)),
        ],
        out_specs=pl.BlockSpec((BE, D), lambda i: (i, 0)),
        out_shape=jax.ShapeDtypeStruct((EPAD, D), jnp.float32),
    )(xd, xs, wa, wb, w1t, b0, b1)


YW = 128
NV = D // LANES


@functools.partial(
    pl.kernel,
    out_type=(
        jax.ShapeDtypeStruct((EPAD, D), jnp.float32),
        jax.ShapeDtypeStruct((EPAD, D), jnp.float32),
    ),
    mesh=_mesh,
    scratch_types=[
        pltpu.VMEM((NCH, CH), jnp.int32),
        pltpu.VMEM((NCH, CH), jnp.int32),
        pltpu.VMEM((EPT + LANES,), jnp.int32),
        pltpu.VMEM((YW, D), jnp.float32),
        pltpu.VMEM((CH, D), jnp.float32),
        pltpu.VMEM((CH, D), jnp.float32),
        pltpu.VMEM((CH, D), jnp.float32),
        pltpu.VMEM((CH, D), jnp.float32),
        pltpu.SemaphoreType.DMA,
        pltpu.SemaphoreType.DMA,
        pltpu.SemaphoreType.DMA,
        pltpu.SemaphoreType.DMA,
        pltpu.SemaphoreType.DMA,
        pltpu.SemaphoreType.DMA,
        pltpu.SemaphoreType.DMA,
        pltpu.SemaphoreType.DMA,
        pltpu.SemaphoreType.DMA,
        pltpu.SemaphoreType.DMA,
        pltpu.SemaphoreType.DMA,
    ],
)
def _sc_gather(y1_hbm, y2_hbm, didx_hbm, sidx_hbm, sdst1d_hbm, gd_hbm, gs_hbm,
               didx_v, sidx_v, dbuf1d, ybuf, bufu0, bufu1, bufv0, bufv1,
               gu0, gu1, gv0, gv1, wu0, wu1, wv0, wv1, si0, si1, sw0):
    wid = lax.axis_index("s") * 2 + lax.axis_index("c")
    row0 = wid * NCH
    base_e = wid * EPT
    c_sidx = pltpu.async_copy(sidx_hbm.at[pl.ds(row0, NCH)], sidx_v, si0)
    c_dbuf = pltpu.async_copy(
        sdst1d_hbm.at[pl.ds(base_e, EPT + LANES)], dbuf1d, si1)
    pltpu.sync_copy(didx_hbm.at[pl.ds(row0, NCH)], didx_v)
    bufu = (bufu0, bufu1)
    bufv = (bufv0, bufv1)
    gu = (gu0, gu1)
    gv = (gv0, gv1)
    wu = (wu0, wu1)
    wv = (wv0, wv1)

    srcg = {}
    c_sidx.wait()
    srcg[0] = pltpu.async_copy(y2_hbm.at[sidx_v.at[0]], bufv[0], gv[0])

    c_dbuf.wait()
    d_lo = dbuf1d[pl.ds(0, LANES)][0]
    d_hi = dbuf1d[pl.ds(EPT - 1, LANES)][0]
    w_lo = jnp.minimum((d_lo // 8) * 8, NPAD - YW)
    fast = (d_hi - w_lo) < YW

    @pl.when(fast)
    def _():
        pltpu.async_copy(y1_hbm.at[pl.ds(w_lo, YW)], ybuf, sw0)

    for j in range(NCH):
        if j == 0:
            @pl.when(fast)
            def _():
                pltpu.make_async_copy(
                    y1_hbm.at[pl.ds(0, YW)], ybuf, sw0).wait()
        b = j & 1
        nb = b ^ 1
        if j + 1 < NCH:
            if j >= 1:
                pltpu.make_async_copy(
                    bufv[nb], gs_hbm.at[pl.ds(0, CH)], wv[nb]).wait()
            srcg[j + 1] = pltpu.async_copy(
                y2_hbm.at[sidx_v.at[j + 1]], bufv[nb], gv[nb])
        if j >= 2:
            pltpu.make_async_copy(
                bufu[b], gd_hbm.at[pl.ds(0, CH)], wu[b]).wait()

        @pl.when(fast)
        def _(jj=j, bb=b):
            @pl.loop(0, CH)
            def _(i):
                dv = dbuf1d[pl.ds(jj * CH + i, LANES)][0]
                lr = dv - w_lo
                for k in range(NV):
                    sl = pl.ds(k * LANES, LANES)
                    bufu[bb][i, sl] = ybuf[lr, sl]

        @pl.when(jnp.logical_not(fast))
        def _(jj=j, bb=b):
            pltpu.async_copy(y1_hbm.at[didx_v.at[jj]], bufu[bb], gu[bb]).wait()

        pltpu.async_copy(bufu[b], gd_hbm.at[pl.ds(base_e + j * CH, CH)], wu[b])
        srcg.pop(j).wait()
        pltpu.async_copy(bufv[b], gs_hbm.at[pl.ds(base_e + j * CH, CH)], wv[b])

    for b in (0, 1):
        pltpu.make_async_copy(bufu[b], gd_hbm.at[pl.ds(0, CH)], wu[b]).wait()
        pltpu.make_async_copy(bufv[b], gs_hbm.at[pl.ds(0, CH)], wv[b]).wait()


@functools.partial(
    pl.kernel,
    out_type=jax.ShapeDtypeStruct((NPAD, D), jnp.float32),
    mesh=_mesh,
    scratch_types=[
        pltpu.VMEM((48,), jnp.int32),
        pltpu.VMEM((SEG + LANES,), jnp.int32),
        pltpu.VMEM((SEG + LANES,), jnp.int32),
        pltpu.VMEM((SEG, D), jnp.float32),
        pltpu.VMEM((SEG, D), jnp.float32),
        pltpu.VMEM((RPT + 16, D), jnp.float32),
        pltpu.SemaphoreType.DMA,
        pltpu.SemaphoreType.DMA,
        pltpu.SemaphoreType.DMA,
        pltpu.SemaphoreType.DMA,
    ],
)
def _sc_segmax(h_hbm, sdst_hbm, starts_hbm, y_hbm,
               starts_v, dbuf0, dbuf1, hbuf0, hbuf1, agg, sd0, sd1, sh0, sh1):
    wid = lax.axis_index("s") * 2 + lax.axis_index("c")
    pltpu.sync_copy(starts_hbm, starts_v)

    zeros = jnp.zeros((LANES,), jnp.float32)

    @pl.loop(0, RPT + 16)
    def _(r):
        for k in range(D // LANES):
            agg[r, pl.ds(k * LANES, LANES)] = zeros

    a = starts_v[pl.ds(wid, LANES)][0]
    b = starts_v[pl.ds(wid + 1, LANES)][0]
    a0 = (a // SEG) * SEG
    nch = (b - a0 + SEG - 1) // SEG
    end64 = a0 + SEG * nch
    base_node = wid * RPT

    def _issue(off, dbuf, hbuf, sdm, shm):
        pltpu.async_copy(sdst_hbm.at[pl.ds(off, SEG)], dbuf.at[pl.ds(0, SEG)], sdm)
        pltpu.async_copy(h_hbm.at[pl.ds(off, SEG)], hbuf, shm)

    def _wait(dbuf, hbuf, sdm, shm):
        pltpu.make_async_copy(sdst_hbm.at[pl.ds(0, SEG)],
                              dbuf.at[pl.ds(0, SEG)], sdm).wait()
        pltpu.make_async_copy(h_hbm.at[pl.ds(0, SEG)], hbuf, shm).wait()

    NV = D // LANES
    neg = jnp.full((LANES,), -3.0e38, jnp.float32)

    def _flush(row, accs):
        for k in range(NV):
            sl = pl.ds(k * LANES, LANES)
            agg[row, sl] = jnp.maximum(agg[row, sl], accs[k])

    def _accum(dbuf, hbuf):
        def body(i, carry):
            cur = carry[0]
            accs = carry[1:]
            dv = dbuf[pl.ds(i, LANES)][0]
            rc = jnp.minimum(jnp.maximum(dv - base_node, -1), RPT) + 8
            nb = rc != cur

            @pl.when(nb)
            def _():
                _flush(cur, accs)

            hrows = [hbuf[i, pl.ds(k * LANES, LANES)] for k in range(NV)]
            new = tuple(
                jnp.where(nb, hrows[k], jnp.maximum(accs[k], hrows[k]))
                for k in range(NV))
            return (rc,) + new

        final = lax.fori_loop(0, SEG, body, (jnp.int32(7),) + (neg,) * NV)
        _flush(final[0], final[1:])

    @pl.when(nch > 0)
    def _():
        _issue(a0, dbuf0, hbuf0, sd0, sh0)

    @pl.loop(0, (nch + 1) // 2)
    def _(t):
        off0 = a0 + t * (2 * SEG)

        @pl.when(off0 + SEG < end64)
        def _():
            _issue(off0 + SEG, dbuf1, hbuf1, sd1, sh1)

        _wait(dbuf0, hbuf0, sd0, sh0)
        _accum(dbuf0, hbuf0)

        @pl.when(off0 + 2 * SEG < end64)
        def _():
            _issue(off0 + 2 * SEG, dbuf0, hbuf0, sd0, sh0)

        @pl.when(off0 + SEG < end64)
        def _():
            _wait(dbuf1, hbuf1, sd1, sh1)
            _accum(dbuf1, hbuf1)

    pltpu.sync_copy(agg.at[pl.ds(8, RPT)], y_hbm.at[pl.ds(wid * RPT, RPT)])


def kernel(x, edge_index, w0_0, b0_0, w1_0, b1_0, w0_1, b0_1, w1_1, b1_1,
           w0_2, b0_2, w1_2, b1_2, w0_3, b0_3, w1_3, b1_3, w0_4, b0_4, w1_4, b1_4):
    src = edge_index[0].astype(jnp.int32)
    dst = edge_index[1].astype(jnp.int32)
    sdst, ssrc = lax.sort((dst, src), num_keys=1)
    starts = jnp.searchsorted(
        sdst, jnp.arange(0, NPAD + RPT, RPT, dtype=jnp.int32)).astype(jnp.int32)
    starts48 = jnp.zeros((48,), jnp.int32).at[:starts.shape[0]].set(starts)
    pad0 = jnp.zeros((EPAD - E,), jnp.int32)
    didx = jnp.concatenate([sdst, pad0]).reshape(EPAD // CH, CH)
    sidx = jnp.concatenate([ssrc, pad0]).reshape(EPAD // CH, CH)
    sdstseg = jnp.concatenate(
        [sdst, jnp.full((EPAD - E + LANES,), NPAD - 8, jnp.int32)])

    y = jnp.zeros((NPAD, D), jnp.float32).at[:N].set(x)
    params = [(w0_0, b0_0, w1_0, b1_0), (w0_1, b0_1, w1_1, b1_1),
              (w0_2, b0_2, w1_2, b1_2), (w0_3, b0_3, w1_3, b1_3),
              (w0_4, b0_4, w1_4, b1_4)]
    for (w0, b0, w1, b1) in params:
        wa = w0[:, :D].T
        wb = w0[:, D:].T
        xd, xs = _sc_gather(y, y, didx, sidx, sdstseg)
        h = _edge_mlp(xd, xs, wa, wb, w1.T, b0.reshape(1, D), b1.reshape(1, D))
        y = _sc_segmax(h, sdstseg, starts48)
    return y[:N]
